# baseline (speedup 1.0000x reference)
import functools

import jax
import jax.numpy as jnp
from jax import lax
from jax.experimental import pallas as pl
from jax.experimental.pallas import tpu as pltpu
from jax.experimental.pallas import tpu_sc as plsc

D = 128
H = 4
HD = D // H
_INV_SQRT_HD = 1.0 / (HD ** 0.5)

NC = 2
NS = 16
L = 16
NW = NC * NS
CHUNK = 64
NACC = 10240


def _dense_body(xs_ref, xt_ref, ts_ref, wsrc_ref, bsrc_ref, wdst_ref, bdst_ref,
                wt_ref, srca_ref, v_ref, dsta_ref, tm_ref):
    xs = xs_ref[...]
    xt = xt_ref[...]
    ts = ts_ref[...]
    rs = jnp.dot(xs, wsrc_ref[...], preferred_element_type=jnp.float32) + bsrc_ref[...]
    srca_ref[...] = rs[:, :3 * D]
    v_ref[...] = rs[:, 3 * D:]
    dsta_ref[...] = jnp.dot(xt, wdst_ref[...], preferred_element_type=jnp.float32) + bdst_ref[...]
    tm_ref[...] = jnp.dot(ts, wt_ref[...], preferred_element_type=jnp.float32)


def _dense_stage(xs, xt, ts, p):
    n = xs.shape[0]
    w1a = p['mlp1_w'][:D, :]
    w1b = p['mlp1_w'][D:, :]
    zd = jnp.zeros((D,), jnp.float32)
    wsrc = jnp.concatenate([w1a, p['eq_w'], p['Wk'], p['Wv']], axis=1)
    bsrc = jnp.concatenate([zd, p['eq_b'], zd, zd])[None, :]
    wdst = jnp.concatenate([w1b, p['ek_w'], p['Wq']], axis=1)
    bdst = jnp.concatenate([p['mlp1_b'], p['ek_b'], zd])[None, :]

    blk = 400
    grid = (n // blk,)
    srca, vtab, dsta, tm = pl.pallas_call(
        _dense_body,
        grid=grid,
        in_specs=[
            pl.BlockSpec((blk, D), lambda i: (i, 0)),
            pl.BlockSpec((blk, D), lambda i: (i, 0)),
            pl.BlockSpec((blk, D), lambda i: (i, 0)),
            pl.BlockSpec((D, 4 * D), lambda i: (0, 0)),
            pl.BlockSpec((1, 4 * D), lambda i: (0, 0)),
            pl.BlockSpec((D, 3 * D), lambda i: (0, 0)),
            pl.BlockSpec((1, 3 * D), lambda i: (0, 0)),
            pl.BlockSpec((D, D), lambda i: (0, 0)),
        ],
        out_specs=[
            pl.BlockSpec((blk, 3 * D), lambda i: (i, 0)),
            pl.BlockSpec((blk, D), lambda i: (i, 0)),
            pl.BlockSpec((blk, 3 * D), lambda i: (i, 0)),
            pl.BlockSpec((blk, D), lambda i: (i, 0)),
        ],
        out_shape=[
            jax.ShapeDtypeStruct((n, 3 * D), jnp.float32),
            jax.ShapeDtypeStruct((n, D), jnp.float32),
            jax.ShapeDtypeStruct((n, 3 * D), jnp.float32),
            jax.ShapeDtypeStruct((n, D), jnp.float32),
        ],
    )(xs, xt, ts, wsrc, bsrc, wdst, bdst, p['Wt'])
    return srca, vtab, dsta, tm


def _sc_pass_a(srca, dsta, sidx, didx, w2, cvec, nch, epw_pad, n_edges):
    mesh = plsc.VectorSubcoreMesh(core_axis_name="c", subcore_axis_name="s")

    @functools.partial(
        pl.kernel,
        out_type=[
            jax.ShapeDtypeStruct((NW, nch, CHUNK, H), jnp.float32),
            jax.ShapeDtypeStruct((NW, nch, CHUNK, H), jnp.float32),
            jax.ShapeDtypeStruct((NW, H, L), jnp.float32),
        ],
        mesh=mesh,
        compiler_params=pltpu.CompilerParams(use_tc_tiling_on_sc=False, needs_layout_passes=False),
        scratch_types=[
            pltpu.VMEM((nch, CHUNK), jnp.int32),
            pltpu.VMEM((nch, CHUNK), jnp.int32),
            pltpu.VMEM((CHUNK, 3 * D), jnp.float32),
            pltpu.VMEM((CHUNK, 3 * D), jnp.float32),
            pltpu.VMEM((CHUNK, H), jnp.float32),
            pltpu.VMEM((CHUNK, H), jnp.float32),
            pltpu.VMEM((D, H), jnp.float32),
            pltpu.VMEM((16,), jnp.float32),
            pltpu.VMEM((H, L), jnp.float32),
            pltpu.SemaphoreType.DMA,
            pltpu.SemaphoreType.DMA,
        ],
    )
    def k(srca_h, dsta_h, sidx_h, didx_h, w2_h, cv_h, p_h, sc_h, zp_h,
          sidx_v, didx_v, sbuf, dbuf, pbuf, scb, w2v, cv, zv, sem1, sem2):
        c = lax.axis_index("c")
        s = lax.axis_index("s")
        w = s * NC + c
        pltpu.sync_copy(sidx_h.at[w], sidx_v)
        pltpu.sync_copy(didx_h.at[w], didx_v)
        pltpu.sync_copy(w2_h, w2v)
        pltpu.sync_copy(cv_h, cv)
        iot = lax.iota(jnp.int32, L)
        zero16 = jnp.zeros((L,), jnp.float32)
        cvv = cv[...]
        b2 = [cvv[i] for i in range(H)]
        sw = cvv[4]
        fw = cvv[5]

        def chunk(j, vz):
            cp1 = pltpu.async_copy(srca_h.at[sidx_v.at[j]], sbuf, sem1)
            cp2 = pltpu.async_copy(dsta_h.at[didx_v.at[j]], dbuf, sem2)
            cp1.wait()
            cp2.wait()
            vzs = list(vz)
            for g in range(CHUNK // L):
                rows = iot + (L * g)
                m = [zero16] * H
                se_l = [None] * H
                sc_l = [None] * H

                for h in range(H):
                    def dbody(dd, car, _h=h):
                        m0, m1, m2, m3, se, sc = car
                        d = dd + _h * HD
                        fd = jnp.full((L,), d, jnp.int32)
                        va = plsc.load_gather(sbuf, [rows, fd])
                        vb = plsc.load_gather(dbuf, [rows, fd])
                        vqe = plsc.load_gather(sbuf, [rows, fd + D])
                        vke = plsc.load_gather(dbuf, [rows, fd + D])
                        vk = plsc.load_gather(sbuf, [rows, fd + 2 * D])
                        vq = plsc.load_gather(dbuf, [rows, fd + 2 * D])
                        hd_ = jnp.maximum(va + vb, 0.0)
                        w0 = plsc.load_gather(w2v, [fd, jnp.full((L,), 0, jnp.int32)])
                        w1 = plsc.load_gather(w2v, [fd, jnp.full((L,), 1, jnp.int32)])
                        w2_ = plsc.load_gather(w2v, [fd, jnp.full((L,), 2, jnp.int32)])
                        w3 = plsc.load_gather(w2v, [fd, jnp.full((L,), 3, jnp.int32)])
                        m0 = m0 + hd_ * w0
                        m1 = m1 + hd_ * w1
                        m2 = m2 + hd_ * w2_
                        m3 = m3 + hd_ * w3
                        se = se + vqe * vke
                        sc = sc + vq * vk
                        return (m0, m1, m2, m3, se, sc)

                    m0, m1, m2, m3, se, sc = lax.fori_loop(
                        0, HD, dbody, (m[0], m[1], m[2], m[3], zero16, zero16))
                    m = [m0, m1, m2, m3]
                    se_l[h] = se
                    sc_l[h] = sc

                valid = (iot + (w * epw_pad + j * CHUNK + L * g)) < n_edges
                for h in range(H):
                    mlpw = 1.0 / (1.0 + jnp.exp(-(m[h] + b2[h])))
                    aw = 1.0 / (1.0 + jnp.exp(-(se_l[h] * _INV_SQRT_HD)))
                    zlog = sw * mlpw + fw * aw
                    ph = jnp.exp(zlog)
                    sch = sc_l[h] * _INV_SQRT_HD
                    vzs[h] = vzs[h] + jnp.where(valid, ph, 0.0)
                    fh = jnp.full((L,), h, jnp.int32)
                    plsc.store_scatter(pbuf, [rows, fh], ph)
                    plsc.store_scatter(scb, [rows, fh], sch)
            pltpu.sync_copy(pbuf, p_h.at[w, j])
            pltpu.sync_copy(scb, sc_h.at[w, j])
            return tuple(vzs)

        vz = lax.fori_loop(0, nch, chunk, (zero16,) * H)
        for h in range(H):
            zv[h, :] = vz[h]
        pltpu.sync_copy(zv, zp_h.at[w])

    return k(srca, dsta, sidx, didx, w2, cvec)


def _sc_pass_b1(p_arr, sc_arr, zp, vtab, sidx, didx, nch):
    mesh = plsc.VectorSubcoreMesh(core_axis_name="c", subcore_axis_name="s")
    rows_per_sub = NACC // NS
    n_zch = rows_per_sub // CHUNK

    @functools.partial(
        pl.kernel,
        out_type=[
            jax.ShapeDtypeStruct((NC, NACC, D), jnp.float32),
            jax.ShapeDtypeStruct((NC, NACC, 8), jnp.float32),
        ],
        mesh=mesh,
        compiler_params=pltpu.CompilerParams(use_tc_tiling_on_sc=False, needs_layout_passes=False),
        scratch_types=[
            pltpu.VMEM((nch, CHUNK), jnp.int32),
            pltpu.VMEM((nch, CHUNK), jnp.int32),
            pltpu.VMEM((CHUNK, H), jnp.float32),
            pltpu.VMEM((CHUNK, H), jnp.float32),
            pltpu.VMEM((CHUNK, 8), jnp.float32),
            pltpu.VMEM((CHUNK, D), jnp.float32),
            pltpu.VMEM((NW, H, L), jnp.float32),
            pltpu.VMEM((CHUNK, D), jnp.float32),
            pltpu.VMEM((CHUNK, 8), jnp.float32),
            pltpu.VMEM_SHARED((NACC, D), jnp.float32),
            pltpu.VMEM_SHARED((NACC, 8), jnp.float32),
            pltpu.SemaphoreType.DMA,
        ],
    )
    def k(p_h, sc_h, zp_h, vtab_h, sidx_h, didx_h, u_h, s_h,
          sidx_v, didx_v, pbuf, scb, ebuf, vbuf, zbuf, zerob, zerob8,
          u_sh, s_sh, sem1):
        c = lax.axis_index("c")
        s = lax.axis_index("s")
        w = s * NC + c
        iot = lax.iota(jnp.int32, L)
        z16 = jnp.zeros((L,), jnp.float32)

        def zfill(i, _):
            for kk in range(D // L):
                zerob[i, pl.ds(kk * L, L)] = z16
            return 0
        lax.fori_loop(0, CHUNK, zfill, 0)

        for g in range(CHUNK // L):
            rws = iot + L * g
            for cc in range(8):
                plsc.store_scatter(zerob8, [rws, jnp.full((L,), cc, jnp.int32)], z16)

        base = s * rows_per_sub
        def zchunk(jj, _):
            pltpu.sync_copy(zerob, u_sh.at[pl.ds(base + jj * CHUNK, CHUNK)])
            pltpu.sync_copy(zerob8, s_sh.at[pl.ds(base + jj * CHUNK, CHUNK)])
            return 0
        lax.fori_loop(0, n_zch, zchunk, 0)
        plsc.subcore_barrier()

        pltpu.sync_copy(zp_h, zbuf)
        zinv = []
        for h in range(H):
            acc = z16
            for ww in range(NW):
                acc = acc + zbuf[ww, h, :]
            zinv.append(1.0 / jnp.full((L,), jnp.sum(acc), jnp.float32))

        for g in range(CHUNK // L):
            rws = iot + L * g
            plsc.store_scatter(ebuf, [rws, jnp.full((L,), 4, jnp.int32)],
                               jnp.full((L,), 1.0, jnp.float32))
            for cc in (5, 6, 7):
                plsc.store_scatter(ebuf, [rws, jnp.full((L,), cc, jnp.int32)], z16)

        pltpu.sync_copy(sidx_h.at[w], sidx_v)
        pltpu.sync_copy(didx_h.at[w], didx_v)

        def chunk(j, _):
            pltpu.sync_copy(p_h.at[w, j], pbuf)
            pltpu.sync_copy(sc_h.at[w, j], scb)
            pltpu.async_copy(vtab_h.at[sidx_v.at[j]], vbuf, sem1).wait()
            for g in range(CHUNK // L):
                rws = iot + L * g
                for h in range(H):
                    fh = jnp.full((L,), h, jnp.int32)
                    pg = plsc.load_gather(pbuf, [rws, fh])
                    sg = plsc.load_gather(scb, [rws, fh])
                    eg = jnp.exp(sg * (pg * zinv[h]))
                    plsc.store_scatter(ebuf, [rws, fh], eg)

            def rowb(r, _):
                fr = jnp.full((L,), r, jnp.int32)
                fs = [plsc.load_gather(ebuf, [fr, jnp.full((L,), hh, jnp.int32)])
                      for hh in range(H)]
                for kk in range(D // L):
                    sl = pl.ds(kk * L, L)
                    vbuf[r, sl] = vbuf[r, sl] * fs[kk // (HD // L)]
                return 0
            lax.fori_loop(0, CHUNK, rowb, 0)
            pltpu.sync_copy(vbuf, u_sh.at[didx_v.at[j]], add=True)
            pltpu.sync_copy(ebuf, s_sh.at[didx_v.at[j]], add=True)
            return 0
        lax.fori_loop(0, nch, chunk, 0)
        plsc.subcore_barrier()

        def ochunk(jj, _):
            sl = pl.ds(base + jj * CHUNK, CHUNK)
            pltpu.sync_copy(u_sh.at[sl], u_h.at[c, sl])
            pltpu.sync_copy(s_sh.at[sl], s_h.at[c, sl])
            return 0
        lax.fori_loop(0, n_zch, ochunk, 0)

    return k(p_arr, sc_arr, zp, vtab, sidx, didx)


def _sc_pass_b2(tm, sidx, didx, nch):
    mesh = plsc.VectorSubcoreMesh(core_axis_name="c", subcore_axis_name="s")
    rows_per_sub = NACC // NS
    n_zch = rows_per_sub // CHUNK

    @functools.partial(
        pl.kernel,
        out_type=[jax.ShapeDtypeStruct((NC, NACC, D), jnp.float32)],
        mesh=mesh,
        compiler_params=pltpu.CompilerParams(use_tc_tiling_on_sc=False, needs_layout_passes=False),
        scratch_types=[
            pltpu.VMEM((nch, CHUNK), jnp.int32),
            pltpu.VMEM((nch, CHUNK), jnp.int32),
            pltpu.VMEM((CHUNK, D), jnp.float32),
            pltpu.VMEM((CHUNK, D), jnp.float32),
            pltpu.VMEM_SHARED((NACC, D), jnp.float32),
            pltpu.SemaphoreType.DMA,
        ],
    )
    def k(tm_h, sidx_h, didx_h, t_h,
          sidx_v, didx_v, tbuf, zerob, t_sh, sem1):
        c = lax.axis_index("c")
        s = lax.axis_index("s")
        w = s * NC + c
        z16 = jnp.zeros((L,), jnp.float32)

        def zfill(i, _):
            for kk in range(D // L):
                zerob[i, pl.ds(kk * L, L)] = z16
            return 0
        lax.fori_loop(0, CHUNK, zfill, 0)

        base = s * rows_per_sub
        def zchunk(jj, _):
            pltpu.sync_copy(zerob, t_sh.at[pl.ds(base + jj * CHUNK, CHUNK)])
            return 0
        lax.fori_loop(0, n_zch, zchunk, 0)
        plsc.subcore_barrier()

        pltpu.sync_copy(sidx_h.at[w], sidx_v)
        pltpu.sync_copy(didx_h.at[w], didx_v)

        def chunk(j, _):
            pltpu.async_copy(tm_h.at[sidx_v.at[j]], tbuf, sem1).wait()
            pltpu.sync_copy(tbuf, t_sh.at[didx_v.at[j]], add=True)
            return 0
        lax.fori_loop(0, nch, chunk, 0)
        plsc.subcore_barrier()

        def ochunk(jj, _):
            sl = pl.ds(base + jj * CHUNK, CHUNK)
            pltpu.sync_copy(t_sh.at[sl], t_h.at[c, sl])
            return 0
        lax.fori_loop(0, n_zch, ochunk, 0)

    return k(tm, sidx, didx)


def _final_body(u_ref, s_ref, t_ref, skx_ref, skt_ref, wout_ref, woutb_ref,
                erep_ref, lng_ref, lnb_ref, sv_ref, x_ref, t_out_ref):
    u = u_ref[0] + u_ref[1]
    s4 = s_ref[0, :, :H] + s_ref[1, :, :H]
    deg = s_ref[0, :, H:H + 1] + s_ref[1, :, H:H + 1]
    t = t_ref[0] + t_ref[1]
    rec = 1.0 / (s4 + 1e-16)
    rec128 = jnp.dot(rec, erep_ref[...], preferred_element_type=jnp.float32)
    x_out = (jnp.dot(u * rec128, wout_ref[...], preferred_element_type=jnp.float32)
             + deg * woutb_ref[...])
    ewa = sv_ref[0, 0]
    rw = sv_ref[0, 1]

    def ln(x):
        mu = jnp.mean(x, axis=-1, keepdims=True)
        xc = x - mu
        var = jnp.mean(xc * xc, axis=-1, keepdims=True)
        return xc / jnp.sqrt(var + 1e-5) * lng_ref[...] + lnb_ref[...]

    x_ref[...] = ln(rw * skx_ref[...] + ewa * jnp.maximum(x_out, 0.0))
    t_out_ref[...] = ln(rw * skt_ref[...] + ewa * jnp.maximum(t, 0.0))


def _final_stage(u2, s2, t2, skip_x, skip_t, p, lng, lnb, ewa, rw):
    n = skip_x.shape[0]
    erep = jnp.repeat(jnp.eye(H, dtype=jnp.float32), HD, axis=1)
    sv = jnp.concatenate([ewa.reshape(1), rw.reshape(1),
                          jnp.zeros((6,), jnp.float32)]).reshape(1, 8)
    blk = 400
    grid = (n // blk,)
    x_tr, t_tr = pl.pallas_call(
        _final_body,
        grid=grid,
        in_specs=[
            pl.BlockSpec((NC, blk, D), lambda i: (0, i, 0)),
            pl.BlockSpec((NC, blk, 8), lambda i: (0, i, 0)),
            pl.BlockSpec((NC, blk, D), lambda i: (0, i, 0)),
            pl.BlockSpec((blk, D), lambda i: (i, 0)),
            pl.BlockSpec((blk, D), lambda i: (i, 0)),
            pl.BlockSpec((D, D), lambda i: (0, 0)),
            pl.BlockSpec((1, D), lambda i: (0, 0)),
            pl.BlockSpec((H, D), lambda i: (0, 0)),
            pl.BlockSpec((1, D), lambda i: (0, 0)),
            pl.BlockSpec((1, D), lambda i: (0, 0)),
            pl.BlockSpec((1, 8), lambda i: (0, 0)),
        ],
        out_specs=[
            pl.BlockSpec((blk, D), lambda i: (i, 0)),
            pl.BlockSpec((blk, D), lambda i: (i, 0)),
        ],
        out_shape=[
            jax.ShapeDtypeStruct((n, D), jnp.float32),
            jax.ShapeDtypeStruct((n, D), jnp.float32),
        ],
    )(u2, s2, t2, skip_x, skip_t, p['Wout_w'], p['Wout_b'][None, :],
      erep, lng[None, :], lnb[None, :], sv)
    return x_tr, t_tr


def _conv(xs, xt, ts, ei, p, lng, lnb, ewa, rw, skip_x, skip_t):
    n = xs.shape[0]
    e = ei.shape[1]
    epw = -(-e // NW)
    nch = -(-epw // CHUNK)
    epw_pad = nch * CHUNK
    tot = NW * epw_pad
    src = ei[0]
    dst = ei[1]
    sidx = jnp.concatenate(
        [src, jnp.zeros((tot - e,), jnp.int32)]).reshape(NW, nch, CHUNK)
    didx = jnp.concatenate(
        [dst, jnp.full((tot - e,), n, jnp.int32)]).reshape(NW, nch, CHUNK)

    srca, vtab, dsta, tm = _dense_stage(xs, xt, ts, p)
    dsta = jnp.concatenate([dsta, jnp.zeros((1, 3 * D), jnp.float32)], axis=0)

    cvec = jnp.concatenate([p['mlp2_b'], p['sw'].reshape(1), p['fw'].reshape(1),
                            jnp.zeros((10,), jnp.float32)])
    p_arr, sc_arr, zp = _sc_pass_a(srca, dsta, sidx, didx, p['mlp2_w'], cvec,
                                   nch, epw_pad, e)
    u2, s2 = _sc_pass_b1(p_arr, sc_arr, zp, vtab, sidx, didx, nch)
    (t2,) = _sc_pass_b2(tm, sidx, didx, nch)

    return _final_stage(u2, s2, t2, skip_x, skip_t, p, lng, lnb, ewa, rw)


def kernel(x_wave, x_transition, x_target, t_wave, t_transition, t_target,
           edge_index_wt, edge_index_tt, params):
    p = params
    x_trans, t_trans = _conv(x_wave, x_transition, t_wave,
                             edge_index_wt, p['c1'], p['ln_g'], p['ln_b'],
                             p['ewa'], p['rw'], x_transition, t_transition)
    x_tgt, t_tgt = _conv(x_trans, x_target, t_trans,
                         edge_index_tt, p['c2'], p['ln_g'], p['ln_b'],
                         p['ewa'], p['rw'], x_target, t_target)
    return (x_wave, x_trans, x_tgt, t_wave, t_trans, t_tgt)
